# Initial kernel scaffold; baseline (speedup 1.0000x reference)
import jax
import jax.numpy as jnp
from jax.experimental import pallas as pl


def kernel(hidden_states, router_w, W1, b1, W2, b2):
    raise NotImplementedError("write your pallas kernel here")


# speedup vs baseline: 4.1444x; 4.1444x over previous
import dataclasses
import functools

import jax
import jax.numpy as jnp
from jax import lax
from jax.experimental import pallas as pl
from jax.experimental.pallas import tpu as pltpu
from jax.experimental.pallas import tpu_sc as plsc

TOPK = 2
TM = 256
_TM_SHIFT = TM.bit_length() - 1


def _sc_compiler_params():
    cp = pltpu.CompilerParams()
    if "needs_layout_passes" in pltpu.CompilerParams.__dataclass_fields__:
        cp = dataclasses.replace(cp, needs_layout_passes=False)
    return cp
NW = 32
NC = 2


def _router_body(flat_ref, rw_ref, idx_ref, prob_ref):
    x = flat_ref[...]
    w = rw_ref[...]
    logits = lax.dot_general(w, x, (((1,), (1,)), ((), ())))
    scores = jax.nn.sigmoid(logits)
    e, t = scores.shape
    row = lax.broadcasted_iota(jnp.int32, (e, t), 0)
    m1 = jnp.max(scores, axis=0, keepdims=True)
    i1 = jnp.min(jnp.where(scores == m1, row, e), axis=0, keepdims=True)
    scores2 = jnp.where(row == i1, -jnp.inf, scores)
    m2 = jnp.max(scores2, axis=0, keepdims=True)
    i2 = jnp.min(jnp.where(scores2 == m2, row, e), axis=0, keepdims=True)
    idx_ref[...] = jnp.concatenate([i1, i2], axis=0).T
    prob_ref[...] = jnp.concatenate([m1, m2], axis=0).T


def _router(flat, router_w):
    t = flat.shape[0]
    return pl.pallas_call(
        _router_body,
        out_shape=(jax.ShapeDtypeStruct((t, TOPK), jnp.int32),
                   jax.ShapeDtypeStruct((t, TOPK), jnp.float32)),
    )(flat, router_w)


def _lane_extract(vec, e):
    return jnp.sum(jnp.where(lax.iota(jnp.int32, 16) == e, vec, 0))


def _make_dispatch(T, H, E, A, Npad, NT):
    chunk = A // NW
    nv = chunk // 16
    mesh = plsc.VectorSubcoreMesh(core_axis_name="c", subcore_axis_name="s")

    @functools.partial(
        pl.kernel,
        out_type=(jax.ShapeDtypeStruct((Npad, H), jnp.float32),
                  jax.ShapeDtypeStruct((A,), jnp.int32),
                  jax.ShapeDtypeStruct((64,), jnp.int32)),
        mesh=mesh,
        compiler_params=_sc_compiler_params(),
        scratch_types=[
            pltpu.VMEM((chunk,), jnp.int32),
            pltpu.VMEM((chunk,), jnp.int32),
            pltpu.VMEM((chunk,), jnp.int32),
            pltpu.VMEM((16,), jnp.int32),
            pltpu.VMEM((NW, 16), jnp.int32),
            pltpu.VMEM_SHARED((NW, 16), jnp.int32),
            pltpu.VMEM((64,), jnp.int32),
            pltpu.VMEM((chunk, H), jnp.float32),
            pltpu.SemaphoreType.DMA,
        ],
    )
    def dispatch(idx_hbm, flat_hbm, xs_hbm, pos_hbm, te_hbm,
                 idx_v, pos_v, tok_v, cnt_v, all_v, shared, te_v, rows_v, sem):
        cid = lax.axis_index("c")
        sid = lax.axis_index("s")
        wid = cid * (NW // NC) + sid
        base = wid * chunk
        iota16 = lax.iota(jnp.int32, 16)

        for half in range(NC):
            gch = half * (NW // NC) + sid
            pltpu.sync_copy(idx_hbm.at[pl.ds(gch * chunk, chunk)], idx_v)
            cnt = jnp.zeros((16,), jnp.int32)
            for e in range(E):
                acc = jnp.zeros((16,), jnp.int32)
                for v in range(nv):
                    m = idx_v[pl.ds(v * 16, 16)] == e
                    acc = acc + plsc.all_reduce_population_count(m)
                cnt = jnp.where(iota16 == e, acc, cnt)
            cnt_v[...] = cnt
            pltpu.sync_copy(cnt_v, shared.at[gch])
        plsc.subcore_barrier()
        pltpu.sync_copy(shared, all_v)
        pltpu.sync_copy(idx_hbm.at[pl.ds(base, chunk)], idx_v)

        wid_v = jnp.zeros((16,), jnp.int32) + wid
        totals = jnp.zeros((16,), jnp.int32)
        before = jnp.zeros((16,), jnp.int32)
        for r in range(NW):
            row = all_v[r, :]
            totals = totals + row
            before = before + jnp.where(wid_v > r, row, 0)

        aligned = ((totals + (TM - 1)) >> _TM_SHIFT) << _TM_SHIFT
        ends = plsc.cumsum(aligned)
        my_base = (ends - aligned) + before

        base_s = [_lane_extract(my_base, e) for e in range(E)]
        end_s = [_lane_extract(ends, e) for e in range(E)]

        carry = [jnp.int32(0)] * E
        for v in range(nv):
            seg = idx_v[pl.ds(v * 16, 16)]
            posv = jnp.zeros((16,), jnp.int32)
            for e in range(E):
                m = seg == e
                mi = m.astype(jnp.int32)
                rk = plsc.cumsum(mi)
                posv = jnp.where(m, base_s[e] + carry[e] + rk - 1, posv)
                carry[e] = carry[e] + jnp.sum(mi)
            pos_v[pl.ds(v * 16, 16)] = posv
        pltpu.sync_copy(pos_v, pos_hbm.at[pl.ds(base, chunk)])

        for v in range(nv):
            tok_v[pl.ds(v * 16, 16)] = (iota16 + (base + v * 16)) // TOPK
        pltpu.async_copy(flat_hbm.at[tok_v], rows_v, sem).wait()
        pltpu.async_copy(rows_v, xs_hbm.at[pos_v], sem).wait()

        @pl.when(wid == 0)
        def _():
            for v in range(4):
                jvec = (iota16 + v * 16) * TM
                te16 = jnp.zeros((16,), jnp.int32)
                for e in range(E):
                    te16 = te16 + (jvec >= end_s[e]).astype(jnp.int32)
                te_v[pl.ds(v * 16, 16)] = jnp.minimum(te16, E - 1)
            pltpu.sync_copy(te_v, te_hbm)

    return dispatch


def _ffn_body(te_ref, xs_ref, w1_ref, b1_ref, w2_ref, b2_ref, out_ref,
              w1b_ref, w2b_ref):
    i = pl.program_id(0)

    @pl.when((i == 0) | (te_ref[i] != te_ref[jnp.maximum(i - 1, 0)]))
    def _():
        w1b_ref[...] = w1_ref[0].astype(jnp.bfloat16)
        w2b_ref[...] = w2_ref[0].astype(jnp.bfloat16)

    x = xs_ref[...].astype(jnp.bfloat16)
    h = lax.dot_general(x, w1b_ref[...], (((1,), (1,)), ((), ())),
                        preferred_element_type=jnp.float32)
    h = h + b1_ref[0]
    h = 0.5 * h * (1.0 + lax.erf(h * 0.7071067811865476))
    o = lax.dot_general(h.astype(jnp.bfloat16), w2b_ref[...],
                        (((1,), (1,)), ((), ())),
                        preferred_element_type=jnp.float32)
    out_ref[...] = o + b2_ref[0]


def _grouped_ffn(te, xs, W1b, b1, W2b, b2, NT):
    npad, hdim = xs.shape
    e, idim, _ = W1b.shape
    grid_spec = pltpu.PrefetchScalarGridSpec(
        num_scalar_prefetch=1,
        grid=(NT,),
        in_specs=[
            pl.BlockSpec((TM, hdim), lambda i, te_ref: (i, 0)),
            pl.BlockSpec((1, idim, hdim), lambda i, te_ref: (te_ref[i], 0, 0)),
            pl.BlockSpec((1, 1, idim), lambda i, te_ref: (te_ref[i], 0, 0)),
            pl.BlockSpec((1, hdim, idim), lambda i, te_ref: (te_ref[i], 0, 0)),
            pl.BlockSpec((1, 1, hdim), lambda i, te_ref: (te_ref[i], 0, 0)),
        ],
        out_specs=pl.BlockSpec((TM, hdim), lambda i, te_ref: (i, 0)),
        scratch_shapes=[pltpu.VMEM((idim, hdim), jnp.bfloat16),
                        pltpu.VMEM((hdim, idim), jnp.bfloat16)],
    )
    return pl.pallas_call(
        _ffn_body,
        grid_spec=grid_spec,
        out_shape=jax.ShapeDtypeStruct((npad, hdim), jnp.float32),
        compiler_params=pltpu.CompilerParams(
            dimension_semantics=("arbitrary",)),
    )(te, xs, W1b, b1, W2b, b2)


def _make_combine_gather(A, H, Npad):
    chunk = A // NW
    mesh = plsc.VectorSubcoreMesh(core_axis_name="c", subcore_axis_name="s")

    @functools.partial(
        pl.kernel,
        out_type=jax.ShapeDtypeStruct((A, H), jnp.float32),
        mesh=mesh,
        compiler_params=_sc_compiler_params(),
        scratch_types=[
            pltpu.VMEM((chunk,), jnp.int32),
            pltpu.VMEM((chunk, H), jnp.float32),
            pltpu.SemaphoreType.DMA,
        ],
    )
    def combine_gather(outs_hbm, pos_hbm, g_hbm, pos_v, rows_v, sem):
        wid = lax.axis_index("c") * (NW // NC) + lax.axis_index("s")
        base = wid * chunk
        pltpu.sync_copy(pos_hbm.at[pl.ds(base, chunk)], pos_v)
        pltpu.async_copy(outs_hbm.at[pos_v], rows_v, sem).wait()
        pltpu.sync_copy(rows_v, g_hbm.at[pl.ds(base, chunk)])

    return combine_gather


def _combine_body(g_ref, p_ref, o_ref):
    g = g_ref[...]
    p = p_ref[...]
    o_ref[...] = g[:, 0, :] * p[:, 0:1] + g[:, 1, :] * p[:, 1:2]


def _combine(g, probs):
    t, _, hdim = g.shape
    bt = 512
    return pl.pallas_call(
        _combine_body,
        grid=(t // bt,),
        in_specs=[pl.BlockSpec((bt, TOPK, hdim), lambda i: (i, 0, 0)),
                  pl.BlockSpec((bt, TOPK), lambda i: (i, 0))],
        out_specs=pl.BlockSpec((bt, hdim), lambda i: (i, 0)),
        out_shape=jax.ShapeDtypeStruct((t, hdim), jnp.float32),
    )(g, probs)


def kernel(hidden_states, router_w, W1, b1, W2, b2):
    b, s, h = hidden_states.shape
    flat = hidden_states.reshape(-1, h)
    t = b * s
    e, idim, _ = W1.shape
    a = t * TOPK
    nt = (a + e * TM) // TM
    npad = nt * TM

    top_idx, top_probs = _router(flat, router_w)
    idx_flat = top_idx.reshape(-1)

    dispatch = _make_dispatch(t, h, e, a, npad, nt)
    xs, pos, te = dispatch(idx_flat, flat)

    out_s = _grouped_ffn(te, xs, W1, b1.reshape(e, 1, idim),
                         W2, b2.reshape(e, 1, h), nt)

    gather = _make_combine_gather(a, h, npad)
    g = gather(out_s, pos)

    out = _combine(g.reshape(t, TOPK, h), top_probs)
    return out.reshape(b, s, h)


# speedup vs baseline: 4.9446x; 1.1931x over previous
import dataclasses
import functools

import jax
import jax.numpy as jnp
from jax import lax
from jax.experimental import pallas as pl
from jax.experimental.pallas import tpu as pltpu
from jax.experimental.pallas import tpu_sc as plsc

TOPK = 2
TM = 256
_TM_SHIFT = TM.bit_length() - 1


def _sc_compiler_params():
    cp = pltpu.CompilerParams()
    if "needs_layout_passes" in pltpu.CompilerParams.__dataclass_fields__:
        cp = dataclasses.replace(cp, needs_layout_passes=False)
    return cp
NW = 32
NC = 2


def _router_body(flat_ref, rw_ref, idx_ref, prob_ref):
    x = flat_ref[...]
    w = rw_ref[...]
    logits = lax.dot_general(w, x, (((1,), (1,)), ((), ())))
    scores = jax.nn.sigmoid(logits)
    e, t = scores.shape
    row = lax.broadcasted_iota(jnp.int32, (e, t), 0)
    m1 = jnp.max(scores, axis=0, keepdims=True)
    i1 = jnp.min(jnp.where(scores == m1, row, e), axis=0, keepdims=True)
    scores2 = jnp.where(row == i1, -jnp.inf, scores)
    m2 = jnp.max(scores2, axis=0, keepdims=True)
    i2 = jnp.min(jnp.where(scores2 == m2, row, e), axis=0, keepdims=True)
    idx_ref[...] = jnp.concatenate([i1, i2], axis=0).T
    prob_ref[...] = jnp.concatenate([m1, m2], axis=0).T


def _router(flat, router_w):
    t = flat.shape[0]
    return pl.pallas_call(
        _router_body,
        out_shape=(jax.ShapeDtypeStruct((t, TOPK), jnp.int32),
                   jax.ShapeDtypeStruct((t, TOPK), jnp.float32)),
    )(flat, router_w)


def _lane_extract(vec, e):
    return jnp.sum(jnp.where(lax.iota(jnp.int32, 16) == e, vec, 0))


def _make_dispatch(T, H, E, A, Npad, NT):
    chunk = A // NW
    nv = chunk // 16
    mesh = plsc.VectorSubcoreMesh(core_axis_name="c", subcore_axis_name="s")

    @functools.partial(
        pl.kernel,
        out_type=(jax.ShapeDtypeStruct((Npad, H), jnp.float32),
                  jax.ShapeDtypeStruct((A,), jnp.int32),
                  jax.ShapeDtypeStruct((64,), jnp.int32)),
        mesh=mesh,
        compiler_params=_sc_compiler_params(),
        scratch_types=[
            pltpu.VMEM((chunk,), jnp.int32),
            pltpu.VMEM((chunk,), jnp.int32),
            pltpu.VMEM((chunk,), jnp.int32),
            pltpu.VMEM((16,), jnp.int32),
            pltpu.VMEM((NW, 16), jnp.int32),
            pltpu.VMEM_SHARED((NW, 16), jnp.int32),
            pltpu.VMEM((64,), jnp.int32),
            pltpu.VMEM((chunk, H), jnp.float32),
            pltpu.SemaphoreType.DMA,
        ],
    )
    def dispatch(idx_hbm, flat_hbm, xs_hbm, pos_hbm, te_hbm,
                 idx_v, pos_v, tok_v, cnt_v, all_v, shared, te_v, rows_v, sem):
        cid = lax.axis_index("c")
        sid = lax.axis_index("s")
        wid = cid * (NW // NC) + sid
        base = wid * chunk
        iota16 = lax.iota(jnp.int32, 16)

        for half in range(NC):
            gch = half * (NW // NC) + sid
            pltpu.sync_copy(idx_hbm.at[pl.ds(gch * chunk, chunk)], idx_v)
            cnt = jnp.zeros((16,), jnp.int32)
            for e in range(E):
                acc = jnp.zeros((16,), jnp.int32)
                for v in range(nv):
                    m = idx_v[pl.ds(v * 16, 16)] == e
                    acc = acc + plsc.all_reduce_population_count(m)
                cnt = jnp.where(iota16 == e, acc, cnt)
            cnt_v[...] = cnt
            pltpu.sync_copy(cnt_v, shared.at[gch])
        plsc.subcore_barrier()
        pltpu.sync_copy(shared, all_v)
        pltpu.sync_copy(idx_hbm.at[pl.ds(base, chunk)], idx_v)

        wid_v = jnp.zeros((16,), jnp.int32) + wid
        totals = jnp.zeros((16,), jnp.int32)
        before = jnp.zeros((16,), jnp.int32)
        for r in range(NW):
            row = all_v[r, :]
            totals = totals + row
            before = before + jnp.where(wid_v > r, row, 0)

        aligned = ((totals + (TM - 1)) >> _TM_SHIFT) << _TM_SHIFT
        ends = plsc.cumsum(aligned)
        my_base = (ends - aligned) + before

        base_s = [_lane_extract(my_base, e) for e in range(E)]
        end_s = [_lane_extract(ends, e) for e in range(E)]

        carry = [jnp.int32(0)] * E
        for v in range(nv):
            seg = idx_v[pl.ds(v * 16, 16)]
            posv = jnp.zeros((16,), jnp.int32)
            for e in range(E):
                m = seg == e
                mi = m.astype(jnp.int32)
                rk = plsc.cumsum(mi)
                posv = jnp.where(m, base_s[e] + carry[e] + rk - 1, posv)
                carry[e] = carry[e] + jnp.sum(mi)
            pos_v[pl.ds(v * 16, 16)] = posv
        pltpu.sync_copy(pos_v, pos_hbm.at[pl.ds(base, chunk)])

        for v in range(nv):
            tok_v[pl.ds(v * 16, 16)] = (iota16 + (base + v * 16)) // TOPK
        pltpu.async_copy(flat_hbm.at[tok_v], rows_v, sem).wait()
        pltpu.async_copy(rows_v, xs_hbm.at[pos_v], sem).wait()

        @pl.when(wid == 0)
        def _():
            for v in range(4):
                jvec = (iota16 + v * 16) * TM
                te16 = jnp.zeros((16,), jnp.int32)
                for e in range(E):
                    te16 = te16 + (jvec >= end_s[e]).astype(jnp.int32)
                te_v[pl.ds(v * 16, 16)] = te16
            pltpu.sync_copy(te_v, te_hbm)

    return dispatch


def _grouped_ffn(te, xs, W1b, b1, W2b, b2, NT):
    npad, hdim = xs.shape
    e, idim, _ = W1b.shape

    def body(te_ref, xs_ref, w1_ref, b1_ref, w2_ref, b2_ref, out_ref,
             w1b_ref, w2b_ref):
        i = pl.program_id(0)
        valid = te_ref[i] < e

        @pl.when(valid & ((i == 0)
                          | (te_ref[i] != te_ref[jnp.maximum(i - 1, 0)])))
        def _():
            w1b_ref[...] = w1_ref[0].astype(jnp.bfloat16)
            w2b_ref[...] = w2_ref[0].astype(jnp.bfloat16)

        @pl.when(valid)
        def _():
            x = xs_ref[...].astype(jnp.bfloat16)
            h = lax.dot_general(x, w1b_ref[...], (((1,), (1,)), ((), ())),
                                preferred_element_type=jnp.float32)
            h = h + b1_ref[0]
            h = 0.5 * h * (1.0 + lax.erf(h * 0.7071067811865476))
            o = lax.dot_general(h.astype(jnp.bfloat16), w2b_ref[...],
                                (((1,), (1,)), ((), ())),
                                preferred_element_type=jnp.float32)
            out_ref[...] = o + b2_ref[0]

    clamp = lambda v: jnp.minimum(v, e - 1)
    grid_spec = pltpu.PrefetchScalarGridSpec(
        num_scalar_prefetch=1,
        grid=(NT,),
        in_specs=[
            pl.BlockSpec((TM, hdim), lambda i, te_ref: (i, 0)),
            pl.BlockSpec((1, idim, hdim),
                         lambda i, te_ref: (clamp(te_ref[i]), 0, 0)),
            pl.BlockSpec((1, 1, idim),
                         lambda i, te_ref: (clamp(te_ref[i]), 0, 0)),
            pl.BlockSpec((1, hdim, idim),
                         lambda i, te_ref: (clamp(te_ref[i]), 0, 0)),
            pl.BlockSpec((1, 1, hdim),
                         lambda i, te_ref: (clamp(te_ref[i]), 0, 0)),
        ],
        out_specs=pl.BlockSpec((TM, hdim), lambda i, te_ref: (i, 0)),
        scratch_shapes=[pltpu.VMEM((idim, hdim), jnp.bfloat16),
                        pltpu.VMEM((hdim, idim), jnp.bfloat16)],
    )
    return pl.pallas_call(
        body,
        grid_spec=grid_spec,
        out_shape=jax.ShapeDtypeStruct((npad, hdim), jnp.float32),
    )(te, xs, W1b, b1, W2b, b2)


def _make_combine(T, H, A, Npad):
    chunk = A // NW
    tchunk = chunk // TOPK
    nh = 2
    hchunk = chunk // nh
    htok = tchunk // nh
    nl = H // 16
    mesh = plsc.VectorSubcoreMesh(core_axis_name="c", subcore_axis_name="s")

    @functools.partial(
        pl.kernel,
        out_type=jax.ShapeDtypeStruct((T, H), jnp.float32),
        mesh=mesh,
        compiler_params=_sc_compiler_params(),
        scratch_types=[
            pltpu.VMEM((hchunk,), jnp.int32),
            pltpu.VMEM((hchunk,), jnp.float32),
            pltpu.VMEM((hchunk, H), jnp.float32),
            pltpu.VMEM((htok, H), jnp.float32),
            pltpu.SemaphoreType.DMA,
        ],
    )
    def combine(outs_hbm, pos_hbm, probs_hbm, o_hbm,
                pos_v, p_v, rows_v, out_v, sem):
        wid = lax.axis_index("c") * (NW // NC) + lax.axis_index("s")
        for half in range(nh):
            abase = wid * chunk + half * hchunk
            tbase = wid * tchunk + half * htok
            pltpu.sync_copy(pos_hbm.at[pl.ds(abase, hchunk)], pos_v)
            pltpu.sync_copy(probs_hbm.at[pl.ds(abase, hchunk)], p_v)
            pltpu.async_copy(outs_hbm.at[pos_v], rows_v, sem).wait()

            zero16 = jnp.zeros((16,), jnp.int32)

            @pl.loop(0, htok)
            def _(j):
                p0 = plsc.load_gather(p_v, [zero16 + 2 * j])
                p1 = plsc.load_gather(p_v, [zero16 + (2 * j + 1)])
                for l in range(nl):
                    sl = pl.ds(l * 16, 16)
                    out_v[j, sl] = (rows_v[2 * j, sl] * p0
                                    + rows_v[2 * j + 1, sl] * p1)

            pltpu.sync_copy(out_v, o_hbm.at[pl.ds(tbase, htok)])

    return combine


def kernel(hidden_states, router_w, W1, b1, W2, b2):
    b, s, h = hidden_states.shape
    flat = hidden_states.reshape(-1, h)
    t = b * s
    e, idim, _ = W1.shape
    a = t * TOPK
    nt = (a + e * TM) // TM
    npad = nt * TM

    top_idx, top_probs = _router(flat, router_w)
    idx_flat = top_idx.reshape(-1)

    dispatch = _make_dispatch(t, h, e, a, npad, nt)
    xs, pos, te = dispatch(idx_flat, flat)

    out_s = _grouped_ffn(te, xs, W1, b1.reshape(e, 1, idim),
                         W2, b2.reshape(e, 1, h), nt)

    combine = _make_combine(t, h, a, npad)
    out = combine(out_s, pos, top_probs.reshape(-1))
    return out.reshape(b, s, h)
